# Initial kernel scaffold; baseline (speedup 1.0000x reference)
import functools

import jax
import jax.numpy as jnp
from jax import lax
from jax.experimental import pallas as pl
from jax.experimental.pallas import tpu as pltpu
from jax.experimental.pallas import tpu_sc as plsc

N_NODES = 10000
D = 128
E = 320000
NW = 32
E_PER_W = E // NW
CHUNK = 80
N_CHUNKS = E_PER_W // CHUNK
N_PAD = 10240
STRIPE = N_PAD // 16
FLUSH = STRIPE // CHUNK

_mesh = plsc.VectorSubcoreMesh(core_axis_name="c", subcore_axis_name="s")


@functools.partial(
    pl.kernel,
    out_type=jax.ShapeDtypeStruct((2 * N_PAD, D), jnp.float32),
    mesh=_mesh,
    scratch_types=[
        pltpu.VMEM((N_CHUNKS, CHUNK), jnp.int32),
        pltpu.VMEM((N_CHUNKS, CHUNK), jnp.int32),
        pltpu.VMEM((CHUNK, D), jnp.float32),
        pltpu.VMEM_SHARED((N_PAD, D), jnp.float32),
        pltpu.SemaphoreType.DMA,
    ],
)
def _gin_scatter(src_hbm, dst_hbm, feat_hbm, out_hbm, sidx, didx, rows, acc, sem):
    cid = lax.axis_index("c")
    sid = lax.axis_index("s")
    wid = cid * 16 + sid

    pltpu.sync_copy(src_hbm.at[pl.ds(wid * N_CHUNKS, N_CHUNKS)], sidx)
    pltpu.sync_copy(dst_hbm.at[pl.ds(wid * N_CHUNKS, N_CHUNKS)], didx)

    zero = jnp.zeros((16,), jnp.float32)

    def zrow(r, _):
        def zcol(c, _):
            rows[r, pl.ds(c * 16, 16)] = zero
            return ()
        lax.fori_loop(0, D // 16, zcol, ())
        return ()

    lax.fori_loop(0, CHUNK, zrow, ())

    def zflush(t, _):
        pltpu.sync_copy(rows, acc.at[pl.ds(sid * STRIPE + t * CHUNK, CHUNK)])
        return ()

    lax.fori_loop(0, FLUSH, zflush, ())
    plsc.subcore_barrier()

    def body(j, _):
        pltpu.async_copy(feat_hbm.at[sidx.at[j]], rows, sem).wait()
        pltpu.sync_copy(rows, acc.at[didx.at[j]], add=True)
        return ()

    lax.fori_loop(0, N_CHUNKS, body, ())
    plsc.subcore_barrier()

    def fbody(t, _):
        r0 = sid * STRIPE + t * CHUNK
        pltpu.sync_copy(acc.at[pl.ds(r0, CHUNK)], rows)
        pltpu.sync_copy(rows, out_hbm.at[pl.ds(cid * N_PAD + r0, CHUNK)])
        return ()

    lax.fori_loop(0, FLUSH, fbody, ())


def _combine_body(eps_ref, feat_ref, p0_ref, p1_ref, out_ref):
    out_ref[...] = ((1.0 + eps_ref[0]) * feat_ref[...]
                    + p0_ref[...] + p1_ref[...])


_R = 80


def _combine(eps, feat, partials):
    return pl.pallas_call(
        _combine_body,
        grid=(N_NODES // _R,),
        in_specs=[
            pl.BlockSpec(memory_space=pltpu.SMEM),
            pl.BlockSpec((_R, D), lambda i: (i, 0)),
            pl.BlockSpec((_R, D), lambda i: (i, 0)),
            pl.BlockSpec((_R, D), lambda i: (i + N_PAD // _R, 0)),
        ],
        out_specs=pl.BlockSpec((_R, D), lambda i: (i, 0)),
        out_shape=jax.ShapeDtypeStruct((N_NODES, D), jnp.float32),
    )(eps, feat, partials, partials)


def kernel(edge_index, split_list, feat, eps):
    src = edge_index[0].astype(jnp.int32).reshape(NW * N_CHUNKS, CHUNK)
    dst = edge_index[1].astype(jnp.int32).reshape(NW * N_CHUNKS, CHUNK)
    partials = _gin_scatter(src, dst, feat)
    return _combine(eps, feat, partials)


# speedup vs baseline: 6.3181x; 6.3181x over previous
import functools

import jax
import jax.numpy as jnp
from jax import lax
from jax.experimental import pallas as pl
from jax.experimental.pallas import tpu as pltpu
from jax.experimental.pallas import tpu_sc as plsc

N_NODES = 10000
D = 128
E = 320000
NW = 32
E_PER_W = E // NW
CHUNK = 80
N_CHUNKS = E_PER_W // CHUNK
N_PAD = 10240
STRIPE = N_PAD // 16
FLUSH = STRIPE // CHUNK

_mesh = plsc.VectorSubcoreMesh(core_axis_name="c", subcore_axis_name="s")


@functools.partial(
    pl.kernel,
    out_type=jax.ShapeDtypeStruct((2 * N_PAD, D), jnp.float32),
    mesh=_mesh,
    scratch_types=[
        pltpu.VMEM((N_CHUNKS, CHUNK), jnp.int32),
        pltpu.VMEM((N_CHUNKS, CHUNK), jnp.int32),
        pltpu.VMEM((CHUNK, D), jnp.float32),
        pltpu.VMEM_SHARED((N_PAD, D), jnp.float32),
        pltpu.SemaphoreType.DMA,
    ],
)
def _gin_scatter(src_hbm, dst_hbm, feat_hbm, out_hbm, sidx, didx, rows, acc, sem):
    cid = lax.axis_index("c")
    sid = lax.axis_index("s")
    wid = cid * 16 + sid

    pltpu.sync_copy(src_hbm.at[wid], sidx)
    pltpu.sync_copy(dst_hbm.at[wid], didx)

    zero = jnp.zeros((16,), jnp.float32)

    def zrow(r, _):
        def zcol(c, _):
            rows[r, pl.ds(c * 16, 16)] = zero
            return ()
        lax.fori_loop(0, D // 16, zcol, ())
        return ()

    lax.fori_loop(0, CHUNK, zrow, ())

    def zflush(t, _):
        pltpu.sync_copy(rows, acc.at[pl.ds(sid * STRIPE + t * CHUNK, CHUNK)])
        return ()

    lax.fori_loop(0, FLUSH, zflush, ())
    plsc.subcore_barrier()

    def body(j, _):
        pltpu.async_copy(feat_hbm.at[sidx.at[j]], rows, sem).wait()
        pltpu.sync_copy(rows, acc.at[didx.at[j]], add=True)
        return ()

    lax.fori_loop(0, N_CHUNKS, body, ())
    plsc.subcore_barrier()

    def fbody(t, _):
        r0 = sid * STRIPE + t * CHUNK
        pltpu.sync_copy(acc.at[pl.ds(r0, CHUNK)], rows)
        pltpu.sync_copy(rows, out_hbm.at[pl.ds(cid * N_PAD + r0, CHUNK)])
        return ()

    lax.fori_loop(0, FLUSH, fbody, ())


def _combine_body(eps_ref, feat_ref, p0_ref, p1_ref, out_ref):
    out_ref[...] = ((1.0 + eps_ref[0]) * feat_ref[...]
                    + p0_ref[...] + p1_ref[...])


_R = 80


def _combine(eps, feat, partials):
    return pl.pallas_call(
        _combine_body,
        grid=(N_NODES // _R,),
        in_specs=[
            pl.BlockSpec(memory_space=pltpu.SMEM),
            pl.BlockSpec((_R, D), lambda i: (i, 0)),
            pl.BlockSpec((_R, D), lambda i: (i, 0)),
            pl.BlockSpec((_R, D), lambda i: (i + N_PAD // _R, 0)),
        ],
        out_specs=pl.BlockSpec((_R, D), lambda i: (i, 0)),
        out_shape=jax.ShapeDtypeStruct((N_NODES, D), jnp.float32),
    )(eps, feat, partials, partials)


def kernel(edge_index, split_list, feat, eps):
    src = edge_index[0].astype(jnp.int32).reshape(NW, N_CHUNKS, CHUNK)
    dst = edge_index[1].astype(jnp.int32).reshape(NW, N_CHUNKS, CHUNK)
    partials = _gin_scatter(src, dst, feat)
    return _combine(eps, feat, partials)


# speedup vs baseline: 9.1117x; 1.4422x over previous
import functools

import jax
import jax.numpy as jnp
from jax import lax
from jax.experimental import pallas as pl
from jax.experimental.pallas import tpu as pltpu
from jax.experimental.pallas import tpu_sc as plsc

N_NODES = 10000
D = 128
E = 320000
NW = 32
E_PER_W = E // NW
CHUNK = 100
N_CHUNKS = E_PER_W // CHUNK
N_STAGES = 2
SCHUNKS = N_CHUNKS // N_STAGES
N_PAD = 10240
STRIPE = N_PAD // 16
FCH = 80
FLUSH = STRIPE // FCH

_mesh = plsc.VectorSubcoreMesh(core_axis_name="c", subcore_axis_name="s")


@functools.partial(
    pl.kernel,
    out_type=jax.ShapeDtypeStruct((2 * N_PAD, D), jnp.float32),
    mesh=_mesh,
    scratch_types=[
        pltpu.VMEM((SCHUNKS, CHUNK), jnp.int32),
        pltpu.VMEM((SCHUNKS, CHUNK), jnp.int32),
        pltpu.VMEM((CHUNK, D), jnp.float32),
        pltpu.VMEM((CHUNK, D), jnp.float32),
        pltpu.VMEM_SHARED((N_PAD, D), jnp.float32),
        pltpu.SemaphoreType.DMA,
        pltpu.SemaphoreType.DMA,
    ],
)
def _gin_scatter(src_hbm, dst_hbm, feat_hbm, out_hbm,
                 sidx, didx, rows0, rows1, acc, sem0, sem1):
    cid = lax.axis_index("c")
    sid = lax.axis_index("s")
    wid = cid * 16 + sid

    zero = jnp.zeros((16,), jnp.float32)

    def zrow(r, _):
        def zcol(c, _):
            rows0[r, pl.ds(c * 16, 16)] = zero
            return ()
        lax.fori_loop(0, D // 16, zcol, ())
        return ()

    lax.fori_loop(0, FCH, zrow, ())

    def zflush(t, _):
        pltpu.sync_copy(rows0.at[pl.ds(0, FCH)],
                        acc.at[pl.ds(sid * STRIPE + t * FCH, FCH)])
        return ()

    lax.fori_loop(0, FLUSH, zflush, ())
    plsc.subcore_barrier()

    for s in range(N_STAGES):
        pltpu.sync_copy(src_hbm.at[wid, s], sidx)
        pltpu.sync_copy(dst_hbm.at[wid, s], didx)
        pltpu.async_copy(feat_hbm.at[sidx.at[0]], rows0, sem0)

        def body(j, _):
            c0 = 2 * j
            c1 = c0 + 1
            pltpu.async_copy(feat_hbm.at[sidx.at[c1]], rows1, sem1)
            pltpu.make_async_copy(feat_hbm.at[sidx.at[c0]], rows0, sem0).wait()
            pltpu.sync_copy(rows0, acc.at[didx.at[c0]], add=True)

            @pl.when(c1 + 1 < SCHUNKS)
            def _():
                pltpu.async_copy(feat_hbm.at[sidx.at[c1 + 1]], rows0, sem0)

            pltpu.make_async_copy(feat_hbm.at[sidx.at[c1]], rows1, sem1).wait()
            pltpu.sync_copy(rows1, acc.at[didx.at[c1]], add=True)
            return ()

        lax.fori_loop(0, SCHUNKS // 2, body, ())
    plsc.subcore_barrier()

    def fbody(t, _):
        r0 = sid * STRIPE + t * FCH
        pltpu.sync_copy(acc.at[pl.ds(r0, FCH)], rows0.at[pl.ds(0, FCH)])
        pltpu.sync_copy(rows0.at[pl.ds(0, FCH)],
                        out_hbm.at[pl.ds(cid * N_PAD + r0, FCH)])
        return ()

    lax.fori_loop(0, FLUSH, fbody, ())


def _combine_body(eps_ref, feat_ref, p0_ref, p1_ref, out_ref):
    out_ref[...] = ((1.0 + eps_ref[0]) * feat_ref[...]
                    + p0_ref[...] + p1_ref[...])


_R = 80


def _combine(eps, feat, partials):
    return pl.pallas_call(
        _combine_body,
        grid=(N_NODES // _R,),
        in_specs=[
            pl.BlockSpec(memory_space=pltpu.SMEM),
            pl.BlockSpec((_R, D), lambda i: (i, 0)),
            pl.BlockSpec((_R, D), lambda i: (i, 0)),
            pl.BlockSpec((_R, D), lambda i: (i + N_PAD // _R, 0)),
        ],
        out_specs=pl.BlockSpec((_R, D), lambda i: (i, 0)),
        out_shape=jax.ShapeDtypeStruct((N_NODES, D), jnp.float32),
    )(eps, feat, partials, partials)


def kernel(edge_index, split_list, feat, eps):
    src = edge_index[0].astype(jnp.int32).reshape(NW, N_STAGES, SCHUNKS, CHUNK)
    dst = edge_index[1].astype(jnp.int32).reshape(NW, N_STAGES, SCHUNKS, CHUNK)
    partials = _gin_scatter(src, dst, feat)
    return _combine(eps, feat, partials)
